# baseline (speedup 1.0000x reference)
import functools
import math

import jax
import jax.numpy as jnp
from jax import lax
from jax.experimental import pallas as pl
from jax.experimental.pallas import tpu as pltpu
from jax.experimental.pallas import tpu_sc as plsc

N = 10000
E = 320000
D = 128
DE = 4
GS = 16
NGV = D - GS
NHMAX = 2000

NPAD = 10240
NHPAD = 2048
DUMP = NHPAD - 1

NC = 2
NS = 16
HW = D // NC
RW = DE * HW
CB1 = 128
CB2 = 64
EPAD = 327680
NCHUNKS = EPAD // CB1
ROWS_PER_TILE = NPAD // NS
HROWS_PER_TILE = NHPAD // NS

_f32 = jnp.float32
_i32 = jnp.int32


def _k1_body(x_ref, w_ref, o_ref):
    o_ref[0] = jnp.dot(x_ref[...], w_ref[0],
                       preferred_element_type=_f32) * (1.0 / math.sqrt(D * DE))


def _k1(x, w1r):
    return pl.pallas_call(
        _k1_body,
        grid=(NC, 10),
        in_specs=[
            pl.BlockSpec((N // 10, D), lambda c, i: (i, 0)),
            pl.BlockSpec((1, D, RW), lambda c, i: (c, 0, 0)),
        ],
        out_specs=pl.BlockSpec((1, N // 10, RW), lambda c, i: (c, i, 0)),
        out_shape=jax.ShapeDtypeStruct((NC, N, RW), _f32),
    )(x, w1r)


def _sc1_body(xp_hbm, src_hbm, dst_hbm, ea_hbm,
              sums_out, cnt_out,
              srcv, dstv, eav, rows, msg, ones, acc_sh, cnt_sh, sem):
    c = lax.axis_index("c")
    s = lax.axis_index("s")
    row0 = s * ROWS_PER_TILE

    def _zfill(i, carry):
        for r in range(HW // 16):
            msg[i, pl.ds(r * 16, 16)] = jnp.zeros((16,), _f32)
        ones[i, pl.ds(0, 16)] = jnp.zeros((16,), _f32)
        return carry
    lax.fori_loop(0, CB1, _zfill, 0)
    for k in range(ROWS_PER_TILE // CB1):
        pltpu.sync_copy(msg, acc_sh.at[pl.ds(row0 + k * CB1, CB1)])
        pltpu.sync_copy(ones, cnt_sh.at[pl.ds(row0 + k * CB1, CB1)])

    def _ones_body(i, carry):
        ones[i, pl.ds(0, 16)] = jnp.full((16,), 1.0, _f32)
        return carry
    lax.fori_loop(0, CB1, _ones_body, 0)
    plsc.subcore_barrier()

    def _edge_body(e, carry):
        ev = eav[pl.ds((e // 4) * 16, 16)]
        lane0 = (e % 4) * DE
        acc = [None] * (HW // 16)
        for j in range(DE):
            w = jnp.take_along_axis(ev, jnp.full((16,), lane0 + j, _i32),
                                    axis=0, mode="promise_in_bounds")
            for r in range(HW // 16):
                seg = rows[e, pl.ds(j * HW + r * 16, 16)]
                acc[r] = seg * w if j == 0 else acc[r] + seg * w
        for r in range(HW // 16):
            msg[e, pl.ds(r * 16, 16)] = acc[r]
        return carry

    def _chunk_body(i, carry):
        ci = s + i * NS
        base = ci * CB1
        pltpu.sync_copy(src_hbm.at[pl.ds(c * EPAD + base, CB1)], srcv)
        pltpu.sync_copy(dst_hbm.at[pl.ds(base, CB1)], dstv)
        pltpu.sync_copy(ea_hbm.at[pl.ds(base * DE, CB1 * DE)],
                        eav.at[pl.ds(0, CB1 * DE)])
        pltpu.async_copy(xp_hbm.at[srcv], rows, sem).wait()
        lax.fori_loop(0, CB1, _edge_body, 0)
        pltpu.sync_copy(msg, acc_sh.at[dstv], add=True)
        pltpu.sync_copy(ones, cnt_sh.at[dstv], add=True)
        return carry

    lax.fori_loop(0, NCHUNKS // NS, _chunk_body, 0)
    plsc.subcore_barrier()

    for k in range(ROWS_PER_TILE // CB1):
        pltpu.sync_copy(acc_sh.at[pl.ds(row0 + k * CB1, CB1)], msg)
        pltpu.sync_copy(msg, sums_out.at[c, pl.ds(row0 + k * CB1, CB1)])
        pltpu.sync_copy(cnt_sh.at[pl.ds(row0 + k * CB1, CB1)], ones)
        pltpu.sync_copy(ones, cnt_out.at[c, pl.ds(row0 + k * CB1, CB1)])


def _sc1(xp2, src2, dst, ea_flat):
    mesh = plsc.VectorSubcoreMesh(core_axis_name="c", subcore_axis_name="s")
    f = functools.partial(
        pl.kernel,
        out_type=(jax.ShapeDtypeStruct((NC, NPAD, HW), _f32),
                  jax.ShapeDtypeStruct((NC, NPAD, 16), _f32)),
        mesh=mesh,
        scratch_types=[
            pltpu.VMEM((CB1,), _i32),
            pltpu.VMEM((CB1,), _i32),
            pltpu.VMEM((CB1 * DE + 16,), _f32),
            pltpu.VMEM((CB1, RW), _f32),
            pltpu.VMEM((CB1, HW), _f32),
            pltpu.VMEM((CB1, 16), _f32),
            pltpu.VMEM_SHARED((NPAD, HW), _f32),
            pltpu.VMEM_SHARED((NPAD, 16), _f32),
            pltpu.SemaphoreType.DMA,
        ],
    )(_sc1_body)
    return f(xp2, src2, dst, ea_flat)


GCH = (NC * EPAD) // CB1 // (NC * NS)


def _sc1a_body(xp_hbm, src_hbm, out_hbm, gv, rowsv, sem):
    wid = lax.axis_index("c") * NS + lax.axis_index("s")

    def _chunk(i, carry):
        base = (wid * GCH + i) * CB1
        pltpu.sync_copy(src_hbm.at[pl.ds(base, CB1)], gv)
        pltpu.async_copy(xp_hbm.at[gv], rowsv, sem).wait()
        pltpu.sync_copy(rowsv, out_hbm.at[pl.ds(base, CB1)])
        return carry
    lax.fori_loop(0, GCH, _chunk, 0)


def _sc1a(xp2, src2):
    mesh = plsc.VectorSubcoreMesh(core_axis_name="c", subcore_axis_name="s")
    f = functools.partial(
        pl.kernel,
        out_type=jax.ShapeDtypeStruct((NC * EPAD, RW), _f32),
        mesh=mesh,
        scratch_types=[
            pltpu.VMEM((CB1,), _i32),
            pltpu.VMEM((CB1, RW), _f32),
            pltpu.SemaphoreType.DMA,
        ],
    )(_sc1a_body)
    return f(xp2, src2)


def _k1b_body(g_ref, ea_ref, o_ref):
    m = g_ref[:, 0 * HW:1 * HW] * ea_ref[:, 0:1]
    for j in range(1, DE):
        m = m + g_ref[:, j * HW:(j + 1) * HW] * ea_ref[:, j:j + 1]
    o_ref[...] = m


def _k1b(grows, ea2):
    blk = 4096
    nb = (NC * EPAD) // blk
    return pl.pallas_call(
        _k1b_body,
        grid=(nb,),
        in_specs=[
            pl.BlockSpec((blk, RW), lambda i: (i, 0)),
            pl.BlockSpec((blk, DE), lambda i: (i, 0)),
        ],
        out_specs=pl.BlockSpec((blk, HW), lambda i: (i, 0)),
        out_shape=jax.ShapeDtypeStruct((NC * EPAD, HW), _f32),
    )(grows, ea2)


def _k2_body(sums_ref, cnt_ref, wg_ref, z_ref, canon_ref,
             xa_ref, xm_ref, ids_ref, gidx_ref):
    ssum = jnp.concatenate([sums_ref[0], sums_ref[1]], axis=1)
    cnt = cnt_ref[0, :, :1]
    node_msg = ssum / jnp.maximum(cnt, 1.0)
    g = jnp.dot(node_msg, wg_ref[...],
                preferred_element_type=_f32) * (1.0 / math.sqrt(D))
    scal = jax.nn.sigmoid(g[:, :GS])
    gates = jnp.tanh(g[:, GS:GS + NGV])
    gated = g[:, GS + NGV:GS + 2 * NGV]
    xa = jnp.concatenate([scal, gates * gated], axis=1)
    heavy = z_ref[...] > 1
    xa_ref[...] = xa
    xm_ref[...] = xa * heavy.astype(_f32)
    ids_ref[...] = jnp.where(heavy, canon_ref[...], DUMP)
    blk = z_ref.shape[0]
    row = (lax.broadcasted_iota(_i32, (blk, 1), 0)
           + pl.program_id(0) * blk + NHPAD)
    gidx_ref[...] = jnp.where(heavy, canon_ref[...], row)


def _k2(sums, cnts, wg, z_pad, canon_pad):
    blk = 1024
    nb = NPAD // blk
    return pl.pallas_call(
        _k2_body,
        grid=(nb,),
        in_specs=[
            pl.BlockSpec((NC, blk, HW), lambda i: (0, i, 0)),
            pl.BlockSpec((1, blk, 16), lambda i: (0, i, 0)),
            pl.BlockSpec((D, GS + 2 * NGV), lambda i: (0, 0)),
            pl.BlockSpec((blk, 1), lambda i: (i, 0)),
            pl.BlockSpec((blk, 1), lambda i: (i, 0)),
        ],
        out_specs=[
            pl.BlockSpec((blk, D), lambda i: (i, 0)),
            pl.BlockSpec((blk, D), lambda i: (i, 0)),
            pl.BlockSpec((blk, 1), lambda i: (i, 0)),
            pl.BlockSpec((blk, 1), lambda i: (i, 0)),
        ],
        out_shape=[
            jax.ShapeDtypeStruct((NPAD, D), _f32),
            jax.ShapeDtypeStruct((NPAD, D), _f32),
            jax.ShapeDtypeStruct((NPAD, 1), _i32),
            jax.ShapeDtypeStruct((NPAD, 1), _i32),
        ],
    )(sums, cnts, wg, z_pad, canon_pad)


def _sc2_body(xm_hbm, ids_hbm,
              hs_out, hc_out,
              xmv, idv, ones2, acc_sh, cnt_sh):
    c = lax.axis_index("c")
    s = lax.axis_index("s")
    row0 = s * HROWS_PER_TILE

    def _zfill(i, carry):
        for r in range(D // 16):
            xmv[i, pl.ds(r * 16, 16)] = jnp.zeros((16,), _f32)
        ones2[i, pl.ds(0, 16)] = jnp.zeros((16,), _f32)
        return carry
    lax.fori_loop(0, CB2, _zfill, 0)
    for k in range(HROWS_PER_TILE // CB2):
        pltpu.sync_copy(xmv, acc_sh.at[pl.ds(row0 + k * CB2, CB2)])
        pltpu.sync_copy(ones2, cnt_sh.at[pl.ds(row0 + k * CB2, CB2)])

    def _ones_body(i, carry):
        ones2[i, pl.ds(0, 16)] = jnp.full((16,), 1.0, _f32)
        return carry
    lax.fori_loop(0, CB2, _ones_body, 0)
    plsc.subcore_barrier()

    for i in range(5):
        base = c * (NPAD // NC) + (s * 5 + i) * CB2
        pltpu.sync_copy(xm_hbm.at[pl.ds(base, CB2)], xmv)
        pltpu.sync_copy(ids_hbm.at[pl.ds(base, CB2)], idv)
        pltpu.sync_copy(xmv, acc_sh.at[idv], add=True)
        pltpu.sync_copy(ones2, cnt_sh.at[idv], add=True)
    plsc.subcore_barrier()

    for k in range(HROWS_PER_TILE // CB2):
        pltpu.sync_copy(acc_sh.at[pl.ds(row0 + k * CB2, CB2)], xmv)
        pltpu.sync_copy(xmv, hs_out.at[c, pl.ds(row0 + k * CB2, CB2)])
        pltpu.sync_copy(cnt_sh.at[pl.ds(row0 + k * CB2, CB2)], ones2)
        pltpu.sync_copy(ones2, hc_out.at[c, pl.ds(row0 + k * CB2, CB2)])


def _sc2(xm, ids_flat):
    mesh = plsc.VectorSubcoreMesh(core_axis_name="c", subcore_axis_name="s")
    f = functools.partial(
        pl.kernel,
        out_type=(jax.ShapeDtypeStruct((NC, NHPAD, D), _f32),
                  jax.ShapeDtypeStruct((NC, NHPAD, 16), _f32)),
        mesh=mesh,
        scratch_types=[
            pltpu.VMEM((CB2, D), _f32),
            pltpu.VMEM((CB2,), _i32),
            pltpu.VMEM((CB2, 16), _f32),
            pltpu.VMEM_SHARED((NHPAD, D), _f32),
            pltpu.VMEM_SHARED((NHPAD, 16), _f32),
        ],
    )(_sc2_body)
    return f(xm, ids_flat)


def _k3_body(hs_ref, hc_ref, w_ref, o_ref):
    ssum = hs_ref[0] + hs_ref[1]
    cnt = hc_ref[0, :, :1] + hc_ref[1, :, :1]
    xh = ssum / jnp.maximum(cnt, 1.0)
    acc = jnp.zeros(o_ref.shape, _f32)
    for j in range(D):
        t = jnp.dot(xh, w_ref[:, j * D:(j + 1) * D], preferred_element_type=_f32)
        acc = acc + xh[:, j:j + 1] * t
    o_ref[...] = acc * (1.0 / D)


def _k3(hsums, hcnts, w2):
    blk = 256
    nb = NHPAD // blk
    return pl.pallas_call(
        _k3_body,
        grid=(nb,),
        in_specs=[
            pl.BlockSpec((NC, blk, D), lambda i: (0, i, 0)),
            pl.BlockSpec((NC, blk, 16), lambda i: (0, i, 0)),
            pl.BlockSpec((D, D * D), lambda i: (0, 0)),
        ],
        out_specs=pl.BlockSpec((blk, D), lambda i: (i, 0)),
        out_shape=jax.ShapeDtypeStruct((NHPAD, D), _f32),
    )(hsums, hcnts, w2)


def _sc3_body(table_hbm, gidx_hbm, out_hbm, gv, rowsv, sem):
    c = lax.axis_index("c")
    s = lax.axis_index("s")
    for i in range(5):
        base = c * (NPAD // NC) + (s * 5 + i) * CB2
        pltpu.sync_copy(gidx_hbm.at[pl.ds(base, CB2)], gv)
        pltpu.async_copy(table_hbm.at[gv], rowsv, sem).wait()
        pltpu.sync_copy(rowsv, out_hbm.at[pl.ds(base, CB2)])


def _sc3(table, gidx_flat):
    mesh = plsc.VectorSubcoreMesh(core_axis_name="c", subcore_axis_name="s")
    f = functools.partial(
        pl.kernel,
        out_type=jax.ShapeDtypeStruct((NPAD, D), _f32),
        mesh=mesh,
        scratch_types=[
            pltpu.VMEM((CB2,), _i32),
            pltpu.VMEM((CB2, D), _f32),
            pltpu.SemaphoreType.DMA,
        ],
    )(_sc3_body)
    return f(table, gidx_flat)


def kernel(x, edge_index, edge_attr, z, canonical, W_msg, W_gate, W_heavy):
    src = jnp.pad(edge_index[0].astype(_i32), (0, EPAD - E))
    dst = jnp.pad(edge_index[1].astype(_i32), (0, EPAD - E),
                  constant_values=NPAD - 1)
    src2 = jnp.concatenate([src, src + N])
    ea_flat = jnp.pad(edge_attr.reshape(-1), (0, (EPAD - E) * DE))
    w1r = W_msg.reshape(D, DE, NC, HW).transpose(2, 0, 1, 3).reshape(NC, D, RW)
    wg = W_gate.reshape(D, GS + 2 * NGV)
    w2 = W_heavy.reshape(D, D * D)
    z_pad = jnp.pad(z.astype(_i32), (0, NPAD - N))
    canon_pad = jnp.pad(canonical.astype(_i32), (0, NPAD - N))

    xp = _k1(x, w1r)
    xp2 = xp.reshape(NC * N, RW)
    grows = _sc1a(xp2, src2)
    ea2 = jnp.concatenate([ea_flat.reshape(EPAD, DE)] * NC, axis=0)
    msg2 = _k1b(grows, ea2)
    s0 = jax.ops.segment_sum(msg2[:E], dst[:E], num_segments=NPAD)
    s1 = jax.ops.segment_sum(msg2[EPAD:EPAD + E], dst[:E], num_segments=NPAD)
    sums = jnp.stack([s0, s1])
    c0 = jax.ops.segment_sum(jnp.ones((E, 1), _f32), dst[:E], num_segments=NPAD)
    cnts = jnp.broadcast_to(c0[None, :, :], (NC, NPAD, 16))
    x_aggr, x_mask, ids, gidx = _k2(sums, cnts, wg,
                                    z_pad[:, None], canon_pad[:, None])
    hs = jax.ops.segment_sum(x_mask, ids.reshape(-1), num_segments=NHPAD)
    hc = jax.ops.segment_sum((ids.reshape(-1) != DUMP).astype(_f32),
                             ids.reshape(-1), num_segments=NHPAD)
    hsums = jnp.stack([hs, jnp.zeros_like(hs)])
    hcnts = jnp.broadcast_to(
        jnp.stack([hc, jnp.zeros_like(hc)])[:, :, None], (NC, NHPAD, 16))
    _unused = _sc1, _sc2
    tp = _k3(hsums, hcnts, w2)
    table = jnp.concatenate([tp, x_aggr], axis=0)
    out_pad = _sc3(table, gidx.reshape(-1))
    return out_pad[:N]
